# Initial kernel scaffold; baseline (speedup 1.0000x reference)
import jax
import jax.numpy as jnp
from jax.experimental import pallas as pl


def kernel(coords_a, coords_b, feat_a, feat_b, W1, b1, W2, b2):
    raise NotImplementedError("write your pallas kernel here")


# speedup vs baseline: 34.6575x; 34.6575x over previous
import functools

import jax
import jax.numpy as jnp
from jax import lax
from jax.experimental import pallas as pl
from jax.experimental.pallas import tpu as pltpu
from jax.experimental.pallas import tpu_sc as plsc

_NA = 8192
_NB = 8192
_D = 256
_K = 8
_BQ = 256
_R = 0.5
_INV_FS2 = 1.0 / (128.0 * 128.0)

_NW = 32
_QPW = _NA // _NW
_CQ = 16


def _pre_body(ca_ref, cbt_ref, fb_ref, w1_ref, idx_ref, w_ref, g_ref):
    a = ca_ref[...] >> 4
    b = cbt_ref[...] >> 4
    d0 = a[:, 0:1] - b[0:1, :]
    d1 = a[:, 1:2] - b[1:2, :]
    d2 = a[:, 2:3] - b[2:3, :]
    dist_i = d0 * d0 + d1 * d1 + d2 * d2
    keys = dist_i * _NB + lax.broadcasted_iota(jnp.int32, (_BQ, _NB), 1)
    cols_i = []
    cols_d = []
    t = keys
    for _ in range(_K):
        m = jnp.min(t, axis=1, keepdims=True)
        t = jnp.where(t == m, jnp.int32(0x7FFFFFFF), t)
        cols_i.append(m & (_NB - 1))
        cols_d.append(m >> 13)
    idx_ref[...] = jnp.concatenate(cols_i, axis=1)
    dist_f = jnp.concatenate(cols_d, axis=1).astype(jnp.float32) * _INV_FS2
    w_ref[...] = _R - jnp.minimum(dist_f, _R)
    g_ref[...] = lax.dot_general(
        fb_ref[...], w1_ref[...], (((1,), (1,)), ((), ())),
        preferred_element_type=jnp.float32,
        precision=lax.Precision.HIGHEST)


def _pre(coords_a, cbt, feat_b, W1):
    grid = _NA // _BQ
    return pl.pallas_call(
        _pre_body,
        grid=(grid,),
        in_specs=[
            pl.BlockSpec((_BQ, 3), lambda i: (i, 0)),
            pl.BlockSpec((3, _NB), lambda i: (0, 0)),
            pl.BlockSpec((_BQ, _D), lambda i: (i, 0)),
            pl.BlockSpec((_D, _D), lambda i: (0, 0)),
        ],
        out_specs=[
            pl.BlockSpec((_BQ, _K), lambda i: (i, 0)),
            pl.BlockSpec((_BQ, _K), lambda i: (i, 0)),
            pl.BlockSpec((_BQ, _D), lambda i: (i, 0)),
        ],
        out_shape=[
            jax.ShapeDtypeStruct((_NA, _K), jnp.int32),
            jax.ShapeDtypeStruct((_NA, _K), jnp.float32),
            jax.ShapeDtypeStruct((_NB, _D), jnp.float32),
        ],
    )(coords_a, cbt, feat_b, W1)


def _combine_kernel(g_hbm, idx_hbm, w_hbm, b1_hbm, s_hbm,
                    idx_v, w_v, rows_v, s_v, b1_v, sem):
    wid = lax.axis_index("s") * 2 + lax.axis_index("c")
    pltpu.sync_copy(b1_hbm, b1_v)
    base_q = wid * _QPW

    def chunk_body(ci, _):
        qb = base_q + ci * _CQ
        pltpu.sync_copy(idx_hbm.at[pl.ds(qb * _K, _CQ * _K)], idx_v)
        pltpu.sync_copy(w_hbm.at[pl.ds(qb * _K, _CQ * _K)], w_v)
        pltpu.async_copy(g_hbm.at[idx_v], rows_v, sem).wait()

        def q_body(q, _):
            wvs = [w_v[q * _K + k, :] for k in range(_K)]
            for c in range(_D // 16):
                b1seg = b1_v[pl.ds(c * 16, 16)]
                acc = jnp.zeros((16,), jnp.float32)
                for k in range(_K):
                    seg = rows_v[q * _K + k, pl.ds(c * 16, 16)]
                    acc = acc + jnp.maximum(wvs[k] * seg + b1seg, 0.0) * wvs[k]
                s_v[q, pl.ds(c * 16, 16)] = acc
            return 0

        lax.fori_loop(0, _CQ, q_body, 0)
        pltpu.sync_copy(s_v, s_hbm.at[pl.ds(qb, _CQ)])
        return 0

    lax.fori_loop(0, _QPW // _CQ, chunk_body, 0)


def _combine(g, idx_flat, w_flat, b1):
    mesh = plsc.VectorSubcoreMesh(core_axis_name="c", subcore_axis_name="s")
    f = functools.partial(
        pl.kernel,
        mesh=mesh,
        out_type=jax.ShapeDtypeStruct((_NA, _D), jnp.float32),
        scratch_types=[
            pltpu.VMEM((_CQ * _K,), jnp.int32),
            pltpu.VMEM((_CQ * _K, 16), jnp.float32),
            pltpu.VMEM((_CQ * _K, _D), jnp.float32),
            pltpu.VMEM((_CQ, _D), jnp.float32),
            pltpu.VMEM((_D,), jnp.float32),
            pltpu.SemaphoreType.DMA,
        ],
    )(_combine_kernel)
    return f(g, idx_flat, w_flat, b1)


def _post_body(fa_ref, s_ref, w2_ref, b2_ref, out_ref):
    out_ref[:, 0:_D] = fa_ref[...]
    out_ref[:, _D:2 * _D] = lax.dot_general(
        s_ref[...], w2_ref[...], (((1,), (1,)), ((), ())),
        preferred_element_type=jnp.float32,
        precision=lax.Precision.HIGHEST) + float(_K) * b2_ref[...]


def _post(feat_a, s, W2, b2):
    grid = _NA // _BQ
    return pl.pallas_call(
        _post_body,
        grid=(grid,),
        in_specs=[
            pl.BlockSpec((_BQ, _D), lambda i: (i, 0)),
            pl.BlockSpec((_BQ, _D), lambda i: (i, 0)),
            pl.BlockSpec((_D, _D), lambda i: (0, 0)),
            pl.BlockSpec((1, _D), lambda i: (0, 0)),
        ],
        out_specs=pl.BlockSpec((_BQ, 2 * _D), lambda i: (i, 0)),
        out_shape=jax.ShapeDtypeStruct((_NA, 2 * _D), jnp.float32),
    )(feat_a, s, W2, b2)


def kernel(coords_a, coords_b, feat_a, feat_b, W1, b1, W2, b2):
    cbt = coords_b.T
    idx, w, g = _pre(coords_a, cbt, feat_b, W1)
    w_exp = jnp.broadcast_to(w.reshape(_NA * _K, 1), (_NA * _K, 16))
    s = _combine(g, idx.reshape(-1), w_exp, b1)
    return _post(feat_a, s, W2, b2.reshape(1, _D))


# speedup vs baseline: 44.0494x; 1.2710x over previous
import functools

import jax
import jax.numpy as jnp
from jax import lax
from jax.experimental import pallas as pl
from jax.experimental.pallas import tpu as pltpu
from jax.experimental.pallas import tpu_sc as plsc

_NA = 8192
_NB = 8192
_D = 256
_K = 8
_BQ = 256
_R = 0.5

_NW = 32
_CQ = 16


def _gmat_body(fb_ref, w1_ref, g_ref):
    g_ref[...] = lax.dot_general(
        fb_ref[...], w1_ref[...], (((1,), (1,)), ((), ())),
        preferred_element_type=jnp.float32,
        precision=lax.Precision.HIGHEST)


def _gmat(feat_b, W1):
    return pl.pallas_call(
        _gmat_body,
        grid=(_NB // _BQ,),
        in_specs=[
            pl.BlockSpec((_BQ, _D), lambda i: (i, 0)),
            pl.BlockSpec((_D, _D), lambda i: (0, 0)),
        ],
        out_specs=pl.BlockSpec((_BQ, _D), lambda i: (i, 0)),
        out_shape=jax.ShapeDtypeStruct((_NB, _D), jnp.float32),
    )(feat_b, W1)


def _topk_body(ca_ref, cbt_ref, idx_ref, w_ref):
    ai = ca_ref[...] >> 4
    bi = cbt_ref[...] >> 4
    a = ai.astype(jnp.bfloat16)
    b = bi.astype(jnp.bfloat16)
    na = (ai[:, 0:1] * ai[:, 0:1] + ai[:, 1:2] * ai[:, 1:2]
          + ai[:, 2:3] * ai[:, 2:3])
    nb = (bi[0:1, :] * bi[0:1, :] + bi[1:2, :] * bi[1:2, :]
          + bi[2:3, :] * bi[2:3, :])
    dot = lax.dot_general(a, b, (((1,), (0,)), ((), ())),
                          preferred_element_type=jnp.float32)
    row_bias = na * _NB
    col_bias = nb * _NB + lax.broadcasted_iota(jnp.int32, (1, _NB), 1)
    keys = dot.astype(jnp.int32) * (-2 * _NB) + row_bias + col_bias
    cols_i = []
    cols_d = []
    m = jnp.min(keys, axis=1, keepdims=True)
    cols_i.append(m & (_NB - 1))
    cols_d.append(m >> 13)
    for _ in range(_K - 1):
        m = jnp.min(jnp.where(keys > m, keys, jnp.int32(0x7FFFFFFF)),
                    axis=1, keepdims=True)
        cols_i.append(m & (_NB - 1))
        cols_d.append(m >> 13)
    idx_ref[...] = jnp.concatenate(cols_i, axis=1)
    dist_f = jnp.concatenate(cols_d, axis=1).astype(jnp.float32) * (
        1.0 / (128.0 * 128.0))
    w8 = _R - jnp.minimum(dist_f, _R)
    w_ref[...] = jnp.concatenate(
        [w8, jnp.zeros((_BQ, 16 - _K), jnp.float32)], axis=1)


def _topk(coords_a_half, cbt):
    nq = coords_a_half.shape[0]
    return pl.pallas_call(
        _topk_body,
        grid=(nq // _BQ,),
        in_specs=[
            pl.BlockSpec((_BQ, 3), lambda i: (i, 0)),
            pl.BlockSpec((3, _NB), lambda i: (0, 0)),
        ],
        out_specs=[
            pl.BlockSpec((_BQ, _K), lambda i: (i, 0)),
            pl.BlockSpec((_BQ, 16), lambda i: (i, 0)),
        ],
        out_shape=[
            jax.ShapeDtypeStruct((nq, _K), jnp.int32),
            jax.ShapeDtypeStruct((nq, 16), jnp.float32),
        ],
    )(coords_a_half, cbt)


def _combine_kernel(nq, g_hbm, idx_hbm, w_hbm, b1_hbm, s_hbm,
                    idx_v, w_v, rows_v, s_v, b1_v, sem):
    wid = lax.axis_index("s") * 2 + lax.axis_index("c")
    qpw = nq // _NW
    pltpu.sync_copy(b1_hbm, b1_v)
    base_q = wid * qpw

    def chunk_body(ci, _):
        qb = base_q + ci * _CQ
        pltpu.sync_copy(idx_hbm.at[pl.ds(qb * _K, _CQ * _K)], idx_v)
        pltpu.sync_copy(w_hbm.at[pl.ds(qb, _CQ)], w_v)
        pltpu.async_copy(g_hbm.at[idx_v], rows_v, sem).wait()

        def q_body(q, _):
            seg = w_v[q, :]
            wvs = [seg.at[jnp.full((16,), k, jnp.int32)]
                      .get(mode="promise_in_bounds")
                   for k in range(_K)]
            for c in range(_D // 16):
                b1seg = b1_v[pl.ds(c * 16, 16)]
                acc = jnp.zeros((16,), jnp.float32)
                for k in range(_K):
                    rseg = rows_v[q * _K + k, pl.ds(c * 16, 16)]
                    acc = acc + jnp.maximum(wvs[k] * rseg + b1seg, 0.0) * wvs[k]
                s_v[q, pl.ds(c * 16, 16)] = acc
            return 0

        lax.fori_loop(0, _CQ, q_body, 0)
        pltpu.sync_copy(s_v, s_hbm.at[pl.ds(qb, _CQ)])
        return 0

    lax.fori_loop(0, qpw // _CQ, chunk_body, 0)


def _combine(g, idx_flat, w16, b1):
    nq = w16.shape[0]
    mesh = plsc.VectorSubcoreMesh(core_axis_name="c", subcore_axis_name="s")
    f = functools.partial(
        pl.kernel, mesh=mesh,
        out_type=jax.ShapeDtypeStruct((nq, _D), jnp.float32),
        scratch_types=[
            pltpu.VMEM((_CQ * _K,), jnp.int32),
            pltpu.VMEM((_CQ, 16), jnp.float32),
            pltpu.VMEM((_CQ * _K, _D), jnp.float32),
            pltpu.VMEM((_CQ, _D), jnp.float32),
            pltpu.VMEM((_D,), jnp.float32),
            pltpu.SemaphoreType.DMA,
        ],
    )(functools.partial(_combine_kernel, nq))
    return f(g, idx_flat, w16, b1)


def _post_body(fa_ref, s_ref, w2_ref, b2_ref, out_ref):
    out_ref[:, 0:_D] = fa_ref[...]
    out_ref[:, _D:2 * _D] = lax.dot_general(
        s_ref[...], w2_ref[...], (((1,), (1,)), ((), ())),
        preferred_element_type=jnp.float32,
        precision=lax.Precision.HIGHEST) + float(_K) * b2_ref[...]


def _post(feat_a_half, s, W2, b2r):
    nq = s.shape[0]
    return pl.pallas_call(
        _post_body,
        grid=(nq // _BQ,),
        in_specs=[
            pl.BlockSpec((_BQ, _D), lambda i: (i, 0)),
            pl.BlockSpec((_BQ, _D), lambda i: (i, 0)),
            pl.BlockSpec((_D, _D), lambda i: (0, 0)),
            pl.BlockSpec((1, _D), lambda i: (0, 0)),
        ],
        out_specs=pl.BlockSpec((_BQ, 2 * _D), lambda i: (i, 0)),
        out_shape=jax.ShapeDtypeStruct((nq, 2 * _D), jnp.float32),
    )(feat_a_half, s, W2, b2r)


def kernel(coords_a, coords_b, feat_a, feat_b, W1, b1, W2, b2):
    cbt = coords_b.T
    b2r = b2.reshape(1, _D)
    h = _NA // 2
    g = _gmat(feat_b, W1)
    idx1, w1 = _topk(coords_a[:h], cbt)
    s1 = _combine(g, idx1.reshape(-1), w1, b1)
    idx2, w2 = _topk(coords_a[h:], cbt)
    s2 = _combine(g, idx2.reshape(-1), w2, b1)
    o1 = _post(feat_a[:h], s1, W2, b2r)
    o2 = _post(feat_a[h:], s2, W2, b2r)
    return jnp.concatenate([o1, o2], axis=0)


# speedup vs baseline: 60.2576x; 1.3680x over previous
import functools

import jax
import jax.numpy as jnp
from jax import lax
from jax.experimental import pallas as pl
from jax.experimental.pallas import tpu as pltpu
from jax.experimental.pallas import tpu_sc as plsc

_NA = 8192
_NB = 8192
_D = 256
_K = 8
_BQ = 256
_R = 0.5

_NW = 32
_CQ = 16


def _gmat_body(fb_ref, w1_ref, g_ref):
    g_ref[...] = lax.dot_general(
        fb_ref[...], w1_ref[...], (((1,), (1,)), ((), ())),
        preferred_element_type=jnp.float32,
        precision=lax.Precision.HIGHEST)


def _gmat(feat_b, W1):
    return pl.pallas_call(
        _gmat_body,
        grid=(_NB // _BQ,),
        in_specs=[
            pl.BlockSpec((_BQ, _D), lambda i: (i, 0)),
            pl.BlockSpec((_D, _D), lambda i: (0, 0)),
        ],
        out_specs=pl.BlockSpec((_BQ, _D), lambda i: (i, 0)),
        out_shape=jax.ShapeDtypeStruct((_NB, _D), jnp.float32),
    )(feat_b, W1)


def _topk_body(ca_ref, cbt_ref, idx_ref, w_ref):
    ai = ca_ref[...] >> 4
    bi = cbt_ref[...] >> 4
    a = ai.astype(jnp.bfloat16)
    b = bi.astype(jnp.bfloat16)
    na = (ai[:, 0:1] * ai[:, 0:1] + ai[:, 1:2] * ai[:, 1:2]
          + ai[:, 2:3] * ai[:, 2:3])
    nb = (bi[0:1, :] * bi[0:1, :] + bi[1:2, :] * bi[1:2, :]
          + bi[2:3, :] * bi[2:3, :])
    dot = lax.dot_general(a, b, (((1,), (0,)), ((), ())),
                          preferred_element_type=jnp.float32)
    row_bias = na * _NB
    col_bias = (nb * _NB + lax.broadcasted_iota(jnp.int32, (1, _NB), 1)
                + jnp.int32(1 << 23))
    keys_i = dot.astype(jnp.int32) * (-2 * _NB) + row_bias + col_bias
    keys = lax.bitcast_convert_type(keys_i, jnp.float32)
    def comp(lst, p, q):
        lo = jnp.minimum(lst[p], lst[q])
        hi = jnp.maximum(lst[p], lst[q])
        lst[p], lst[q] = lo, hi

    def merge8(a, b):
        m8 = [jnp.minimum(a[i], b[7 - i]) for i in range(8)]
        for p, q in ((0, 4), (1, 5), (2, 6), (3, 7),
                     (0, 2), (1, 3), (4, 6), (5, 7),
                     (0, 1), (2, 3), (4, 5), (6, 7)):
            comp(m8, p, q)
        return m8

    net8 = ((0, 1), (2, 3), (4, 5), (6, 7),
            (0, 2), (1, 3), (4, 6), (5, 7),
            (1, 2), (5, 6),
            (0, 4), (1, 5), (2, 6), (3, 7),
            (2, 4), (3, 5),
            (1, 2), (3, 4), (5, 6))
    lists = []
    for j in range(8):
        grp = [keys[:, (8 * j + i) * 128:(8 * j + i + 1) * 128]
               for i in range(8)]
        for p, q in net8:
            comp(grp, p, q)
        lists.append(grp)
    while len(lists) > 1:
        lists = [merge8(lists[i], lists[i + 1])
                 for i in range(0, len(lists), 2)]
    cur = lists[0]
    w = 128
    while w > 1:
        w //= 2
        cur = merge8([x[:, :w] for x in cur], [x[:, w:] for x in cur])
    mi = [lax.bitcast_convert_type(m, jnp.int32) - jnp.int32(1 << 23)
          for m in cur]
    idx_ref[...] = jnp.concatenate([m & (_NB - 1) for m in mi], axis=1)
    dist_f = jnp.concatenate([m >> 13 for m in mi],
                             axis=1).astype(jnp.float32) * (
        1.0 / (128.0 * 128.0))
    w8 = _R - jnp.minimum(dist_f, _R)
    w_ref[...] = jnp.concatenate(
        [w8, jnp.zeros((_BQ, 16 - _K), jnp.float32)], axis=1)


def _topk(coords_a_half, cbt):
    nq = coords_a_half.shape[0]
    return pl.pallas_call(
        _topk_body,
        grid=(nq // _BQ,),
        in_specs=[
            pl.BlockSpec((_BQ, 3), lambda i: (i, 0)),
            pl.BlockSpec((3, _NB), lambda i: (0, 0)),
        ],
        out_specs=[
            pl.BlockSpec((_BQ, _K), lambda i: (i, 0)),
            pl.BlockSpec((_BQ, 16), lambda i: (i, 0)),
        ],
        out_shape=[
            jax.ShapeDtypeStruct((nq, _K), jnp.int32),
            jax.ShapeDtypeStruct((nq, 16), jnp.float32),
        ],
    )(coords_a_half, cbt)


def _combine_kernel(nq, g_hbm, idx_hbm, w_hbm, b1_hbm, s_hbm,
                    idx_v, w_v, rows_v, s_v, b1_v, sem):
    wid = lax.axis_index("s") * 2 + lax.axis_index("c")
    qpw = nq // _NW
    pltpu.sync_copy(b1_hbm, b1_v)
    base_q = wid * qpw

    def chunk_body(ci, _):
        qb = base_q + ci * _CQ
        pltpu.sync_copy(idx_hbm.at[pl.ds(qb * _K, _CQ * _K)], idx_v)
        pltpu.sync_copy(w_hbm.at[pl.ds(qb, _CQ)], w_v)
        pltpu.async_copy(g_hbm.at[idx_v], rows_v, sem).wait()

        def q_body(q, _):
            seg = w_v[q, :]
            wvs = [seg.at[jnp.full((16,), k, jnp.int32)]
                      .get(mode="promise_in_bounds")
                   for k in range(_K)]
            for c in range(_D // 16):
                b1seg = b1_v[pl.ds(c * 16, 16)]
                acc = jnp.zeros((16,), jnp.float32)
                for k in range(_K):
                    rseg = rows_v[q * _K + k, pl.ds(c * 16, 16)]
                    acc = acc + jnp.maximum(wvs[k] * rseg + b1seg, 0.0) * wvs[k]
                s_v[q, pl.ds(c * 16, 16)] = acc
            return 0

        lax.fori_loop(0, _CQ, q_body, 0)
        pltpu.sync_copy(s_v, s_hbm.at[pl.ds(qb, _CQ)])
        return 0

    lax.fori_loop(0, qpw // _CQ, chunk_body, 0)


def _combine(g, idx_flat, w16, b1):
    nq = w16.shape[0]
    mesh = plsc.VectorSubcoreMesh(core_axis_name="c", subcore_axis_name="s")
    f = functools.partial(
        pl.kernel, mesh=mesh,
        out_type=jax.ShapeDtypeStruct((nq, _D), jnp.float32),
        scratch_types=[
            pltpu.VMEM((_CQ * _K,), jnp.int32),
            pltpu.VMEM((_CQ, 16), jnp.float32),
            pltpu.VMEM((_CQ * _K, _D), jnp.float32),
            pltpu.VMEM((_CQ, _D), jnp.float32),
            pltpu.VMEM((_D,), jnp.float32),
            pltpu.SemaphoreType.DMA,
        ],
    )(functools.partial(_combine_kernel, nq))
    return f(g, idx_flat, w16, b1)


def _post_body(fa_ref, s_ref, w2_ref, b2_ref, out_ref):
    out_ref[:, 0:_D] = fa_ref[...]
    out_ref[:, _D:2 * _D] = lax.dot_general(
        s_ref[...], w2_ref[...], (((1,), (1,)), ((), ())),
        preferred_element_type=jnp.float32,
        precision=lax.Precision.HIGHEST) + float(_K) * b2_ref[...]


def _post(feat_a_half, s, W2, b2r):
    nq = s.shape[0]
    return pl.pallas_call(
        _post_body,
        grid=(nq // _BQ,),
        in_specs=[
            pl.BlockSpec((_BQ, _D), lambda i: (i, 0)),
            pl.BlockSpec((_BQ, _D), lambda i: (i, 0)),
            pl.BlockSpec((_D, _D), lambda i: (0, 0)),
            pl.BlockSpec((1, _D), lambda i: (0, 0)),
        ],
        out_specs=pl.BlockSpec((_BQ, 2 * _D), lambda i: (i, 0)),
        out_shape=jax.ShapeDtypeStruct((nq, 2 * _D), jnp.float32),
    )(feat_a_half, s, W2, b2r)


def kernel(coords_a, coords_b, feat_a, feat_b, W1, b1, W2, b2):
    cbt = coords_b.T
    b2r = b2.reshape(1, _D)
    h = _NA // 2
    g = _gmat(feat_b, W1)
    idx1, w1 = _topk(coords_a[:h], cbt)
    s1 = _combine(g, idx1.reshape(-1), w1, b1)
    idx2, w2 = _topk(coords_a[h:], cbt)
    s2 = _combine(g, idx2.reshape(-1), w2, b1)
    o1 = _post(feat_a[:h], s1, W2, b2r)
    o2 = _post(feat_a[h:], s2, W2, b2r)
    return jnp.concatenate([o1, o2], axis=0)


# speedup vs baseline: 69.3497x; 1.1509x over previous
import functools

import jax
import jax.numpy as jnp
from jax import lax
from jax.experimental import pallas as pl
from jax.experimental.pallas import tpu as pltpu
from jax.experimental.pallas import tpu_sc as plsc

_NA = 8192
_NB = 8192
_D = 256
_K = 8
_BQ = 512
_R = 0.5

_NW = 32
_CQ = 16


def _gmat_body(fb_ref, w1_ref, g_ref):
    g_ref[...] = lax.dot_general(
        fb_ref[...], w1_ref[...], (((1,), (1,)), ((), ())),
        preferred_element_type=jnp.float32,
        precision=lax.Precision.HIGHEST)


def _gmat(feat_b, W1):
    return pl.pallas_call(
        _gmat_body,
        grid=(_NB // _BQ,),
        in_specs=[
            pl.BlockSpec((_BQ, _D), lambda i: (i, 0)),
            pl.BlockSpec((_D, _D), lambda i: (0, 0)),
        ],
        out_specs=pl.BlockSpec((_BQ, _D), lambda i: (i, 0)),
        out_shape=jax.ShapeDtypeStruct((_NB, _D), jnp.float32),
    )(feat_b, W1)


def _topk_body(ca_ref, cbt_ref, idx_ref, w_ref):
    ai = ca_ref[...] >> 4
    bi = cbt_ref[...] >> 4
    a = ai.astype(jnp.bfloat16)
    b = bi.astype(jnp.bfloat16)
    na = (ai[:, 0:1] * ai[:, 0:1] + ai[:, 1:2] * ai[:, 1:2]
          + ai[:, 2:3] * ai[:, 2:3])
    nb = (bi[0:1, :] * bi[0:1, :] + bi[1:2, :] * bi[1:2, :]
          + bi[2:3, :] * bi[2:3, :])
    dot = lax.dot_general(a, b, (((1,), (0,)), ((), ())),
                          preferred_element_type=jnp.float32)
    row_bias = na * _NB
    col_bias = (nb * _NB + lax.broadcasted_iota(jnp.int32, (1, _NB), 1)
                + jnp.int32(1 << 23))
    keys_i = dot.astype(jnp.int32) * (-2 * _NB) + row_bias + col_bias
    keys = lax.bitcast_convert_type(keys_i, jnp.float32)
    def comp(lst, p, q):
        lo = jnp.minimum(lst[p], lst[q])
        hi = jnp.maximum(lst[p], lst[q])
        lst[p], lst[q] = lo, hi

    def merge8(a, b):
        m8 = [jnp.minimum(a[i], b[7 - i]) for i in range(8)]
        for p, q in ((0, 4), (1, 5), (2, 6), (3, 7),
                     (0, 2), (1, 3), (4, 6), (5, 7),
                     (0, 1), (2, 3), (4, 5), (6, 7)):
            comp(m8, p, q)
        return m8

    net8 = ((0, 1), (2, 3), (4, 5), (6, 7),
            (0, 2), (1, 3), (4, 6), (5, 7),
            (1, 2), (5, 6),
            (0, 4), (1, 5), (2, 6), (3, 7),
            (2, 4), (3, 5),
            (1, 2), (3, 4), (5, 6))
    lists = []
    for j in range(8):
        grp = [keys[:, (8 * j + i) * 128:(8 * j + i + 1) * 128]
               for i in range(8)]
        for p, q in net8:
            comp(grp, p, q)
        lists.append(grp)
    while len(lists) > 1:
        lists = [merge8(lists[i], lists[i + 1])
                 for i in range(0, len(lists), 2)]
    cur = lists[0]
    w = 128
    while w > 1:
        w //= 2
        cur = merge8([x[:, :w] for x in cur], [x[:, w:] for x in cur])
    mi = [lax.bitcast_convert_type(m, jnp.int32) - jnp.int32(1 << 23)
          for m in cur]
    idx_ref[...] = jnp.concatenate([m & (_NB - 1) for m in mi], axis=1)
    dist_f = jnp.concatenate([m >> 13 for m in mi],
                             axis=1).astype(jnp.float32) * (
        1.0 / (128.0 * 128.0))
    w8 = _R - jnp.minimum(dist_f, _R)
    w_ref[...] = jnp.concatenate(
        [w8, jnp.zeros((_BQ, 16 - _K), jnp.float32)], axis=1)


def _topk(coords_a_half, cbt):
    nq = coords_a_half.shape[0]
    return pl.pallas_call(
        _topk_body,
        grid=(nq // _BQ,),
        in_specs=[
            pl.BlockSpec((_BQ, 3), lambda i: (i, 0)),
            pl.BlockSpec((3, _NB), lambda i: (0, 0)),
        ],
        out_specs=[
            pl.BlockSpec((_BQ, _K), lambda i: (i, 0)),
            pl.BlockSpec((_BQ, 16), lambda i: (i, 0)),
        ],
        out_shape=[
            jax.ShapeDtypeStruct((nq, _K), jnp.int32),
            jax.ShapeDtypeStruct((nq, 16), jnp.float32),
        ],
    )(coords_a_half, cbt)


def _combine_kernel(nq, g_hbm, idx_hbm, w_hbm, b1_hbm, s_hbm,
                    idx_v, w_v, rows_v, s_v, b1_v, sem):
    wid = lax.axis_index("s") * 2 + lax.axis_index("c")
    qpw = nq // _NW
    pltpu.sync_copy(b1_hbm, b1_v)
    base_q = wid * qpw

    def chunk_body(ci, _):
        qb = base_q + ci * _CQ
        pltpu.sync_copy(idx_hbm.at[pl.ds(qb * _K, _CQ * _K)], idx_v)
        pltpu.sync_copy(w_hbm.at[pl.ds(qb, _CQ)], w_v)
        pltpu.async_copy(g_hbm.at[idx_v], rows_v, sem).wait()

        def q_body(q, _):
            seg = w_v[q, :]
            wvs = [seg.at[jnp.full((16,), k, jnp.int32)]
                      .get(mode="promise_in_bounds")
                   for k in range(_K)]
            for c in range(_D // 16):
                b1seg = b1_v[pl.ds(c * 16, 16)]
                acc = jnp.zeros((16,), jnp.float32)
                for k in range(_K):
                    rseg = rows_v[q * _K + k, pl.ds(c * 16, 16)]
                    acc = acc + jnp.maximum(wvs[k] * rseg + b1seg, 0.0) * wvs[k]
                s_v[q, pl.ds(c * 16, 16)] = acc
            return 0

        lax.fori_loop(0, _CQ, q_body, 0)
        pltpu.sync_copy(s_v, s_hbm.at[pl.ds(qb, _CQ)])
        return 0

    lax.fori_loop(0, qpw // _CQ, chunk_body, 0)


def _combine(g, idx_flat, w16, b1):
    nq = w16.shape[0]
    mesh = plsc.VectorSubcoreMesh(core_axis_name="c", subcore_axis_name="s")
    f = functools.partial(
        pl.kernel, mesh=mesh,
        out_type=jax.ShapeDtypeStruct((nq, _D), jnp.float32),
        scratch_types=[
            pltpu.VMEM((_CQ * _K,), jnp.int32),
            pltpu.VMEM((_CQ, 16), jnp.float32),
            pltpu.VMEM((_CQ * _K, _D), jnp.float32),
            pltpu.VMEM((_CQ, _D), jnp.float32),
            pltpu.VMEM((_D,), jnp.float32),
            pltpu.SemaphoreType.DMA,
        ],
    )(functools.partial(_combine_kernel, nq))
    return f(g, idx_flat, w16, b1)


def _post_body(fa_ref, s_ref, w2_ref, b2_ref, out_ref):
    out_ref[:, 0:_D] = fa_ref[...]
    out_ref[:, _D:2 * _D] = lax.dot_general(
        s_ref[...], w2_ref[...], (((1,), (1,)), ((), ())),
        preferred_element_type=jnp.float32,
        precision=lax.Precision.HIGHEST) + float(_K) * b2_ref[...]


def _post(feat_a_half, s, W2, b2r):
    nq = s.shape[0]
    return pl.pallas_call(
        _post_body,
        grid=(nq // _BQ,),
        in_specs=[
            pl.BlockSpec((_BQ, _D), lambda i: (i, 0)),
            pl.BlockSpec((_BQ, _D), lambda i: (i, 0)),
            pl.BlockSpec((_D, _D), lambda i: (0, 0)),
            pl.BlockSpec((1, _D), lambda i: (0, 0)),
        ],
        out_specs=pl.BlockSpec((_BQ, 2 * _D), lambda i: (i, 0)),
        out_shape=jax.ShapeDtypeStruct((nq, 2 * _D), jnp.float32),
    )(feat_a_half, s, W2, b2r)


def kernel(coords_a, coords_b, feat_a, feat_b, W1, b1, W2, b2):
    cbt = coords_b.T
    b2r = b2.reshape(1, _D)
    g = _gmat(feat_b, W1)
    h = _NA // 4
    outs = []
    for p_ in range(4):
        sl = slice(p_ * h, (p_ + 1) * h)
        idx_p, w_p = _topk(coords_a[sl], cbt)
        s_p = _combine(g, idx_p.reshape(-1), w_p, b1)
        outs.append(_post(feat_a[sl], s_p, W2, b2r))
    return jnp.concatenate(outs, axis=0)
